# baseline (speedup 1.0000x reference)
import functools

import jax
import jax.numpy as jnp
from jax import lax
from jax.experimental import pallas as pl
from jax.experimental.pallas import tpu as pltpu
from jax.experimental.pallas import tpu_sc as plsc

N_NODES = 10000
N_EDGES = 160000
D_IN = 256
D_OUT = 256

HALF = D_IN // 2
NC = 2
NS = 16
CHUNK = 128
CHUNKS_PER_TILE = 83
EDGES_PER_TILE = CHUNKS_PER_TILE * CHUNK
E_PAD = NS * EDGES_PER_TILE
ROWS_PER_TILE = 632
N_PAD = NS * ROWS_PER_TILE
ACC_ROWS = N_PAD


def _sc_scatter(feat_cat, src2, dst_r):
    @functools.partial(
        pl.kernel,
        out_type=jax.ShapeDtypeStruct((NC, N_PAD, HALF), jnp.float32),
        mesh=plsc.VectorSubcoreMesh(core_axis_name="c", subcore_axis_name="s"),
        scratch_types=[
            pltpu.VMEM_SHARED((ACC_ROWS, HALF), jnp.float32),
            pltpu.VMEM((CHUNKS_PER_TILE, CHUNK), jnp.int32),
            pltpu.VMEM((CHUNKS_PER_TILE, CHUNK), jnp.int32),
            pltpu.VMEM((CHUNK, HALF), jnp.float32),
            pltpu.SemaphoreType.DMA,
        ],
    )
    def k(feat_hbm, src_hbm, dst_hbm, h_hbm, acc, src_v, dst_v, rows_v, sem):
        c = lax.axis_index("c")
        s = lax.axis_index("s")
        row0 = s * ROWS_PER_TILE
        pltpu.sync_copy(
            feat_hbm.at[pl.ds(c * N_PAD + row0, ROWS_PER_TILE)],
            acc.at[pl.ds(row0, ROWS_PER_TILE)],
        )
        pltpu.sync_copy(src_hbm.at[c, s], src_v)
        pltpu.sync_copy(dst_hbm.at[s], dst_v)
        plsc.subcore_barrier()

        @pl.loop(0, CHUNKS_PER_TILE)
        def _(j):
            pltpu.async_copy(feat_hbm.at[src_v.at[j]], rows_v, sem).wait()
            pltpu.sync_copy(rows_v, acc.at[dst_v.at[j]], add=True)

        plsc.subcore_barrier()
        pltpu.sync_copy(
            acc.at[pl.ds(row0, ROWS_PER_TILE)],
            h_hbm.at[c, pl.ds(row0, ROWS_PER_TILE)],
        )

    return k(feat_cat, src2, dst_r)


ROW_BLK = 1000


def _mm_body(h0_ref, h1_ref, wl_ref, wr_ref, b_ref, o_ref):
    o_ref[...] = (
        jnp.dot(h0_ref[0], wl_ref[...], preferred_element_type=jnp.float32,
                precision=lax.Precision.HIGHEST)
        + jnp.dot(h1_ref[0], wr_ref[...], preferred_element_type=jnp.float32,
                  precision=lax.Precision.HIGHEST)
        + b_ref[...]
    )


def _tc_linear(h2, W, b):
    wl = W[:, :HALF].T
    wr = W[:, HALF:].T
    b2 = b.reshape(1, D_OUT)
    return pl.pallas_call(
        _mm_body,
        grid=(N_NODES // ROW_BLK,),
        in_specs=[
            pl.BlockSpec((1, ROW_BLK, HALF), lambda i: (0, i, 0)),
            pl.BlockSpec((1, ROW_BLK, HALF), lambda i: (1, i, 0)),
            pl.BlockSpec((HALF, D_OUT), lambda i: (0, 0)),
            pl.BlockSpec((HALF, D_OUT), lambda i: (0, 0)),
            pl.BlockSpec((1, D_OUT), lambda i: (0, 0)),
        ],
        out_specs=pl.BlockSpec((ROW_BLK, D_OUT), lambda i: (i, 0)),
        out_shape=jax.ShapeDtypeStruct((N_NODES, D_OUT), jnp.float32),
    )(h2, h2, wl, wr, b2)


@jax.jit
def kernel(feature, edge_index, W, b):
    src = edge_index[0].astype(jnp.int32)
    dst = edge_index[1].astype(jnp.int32)
    pad = E_PAD - N_EDGES
    src_p = jnp.concatenate([src, jnp.zeros((pad,), jnp.int32)])
    dst_p = jnp.concatenate(
        [dst, N_NODES + (jnp.arange(pad, dtype=jnp.int32) % (N_PAD - N_NODES))])
    src_r = src_p.reshape(NS, CHUNKS_PER_TILE, CHUNK)
    src2 = jnp.stack([src_r, src_r + N_PAD])
    dst_r = dst_p.reshape(NS, CHUNKS_PER_TILE, CHUNK)
    feat_pad = jnp.concatenate(
        [feature, jnp.zeros((N_PAD - N_NODES, D_IN), jnp.float32)])
    feat_cat = feat_pad.reshape(N_PAD, NC, HALF).transpose(1, 0, 2).reshape(
        NC * N_PAD, HALF)

    h2 = _sc_scatter(feat_cat, src2, dst_r)
    return _tc_linear(h2, W, b)


# speedup vs baseline: 3.0090x; 3.0090x over previous
import functools

import jax
import jax.numpy as jnp
from jax import lax
from jax.experimental import pallas as pl
from jax.experimental.pallas import tpu as pltpu
from jax.experimental.pallas import tpu_sc as plsc

N_NODES = 10000
N_EDGES = 160000
D_IN = 256
D_OUT = 256

HALF = D_IN // 2
NC = 2
NS = 16
CHUNK = 128
CHUNKS_PER_TILE = 83
EDGES_PER_TILE = CHUNKS_PER_TILE * CHUNK
E_PAD = NS * EDGES_PER_TILE
ROWS_PER_TILE = 632
N_PAD = NS * ROWS_PER_TILE
ACC_ROWS = N_PAD


def _sc_scatter(feat_cat, src2, dst_r):
    @functools.partial(
        pl.kernel,
        out_type=jax.ShapeDtypeStruct((NC, N_PAD, HALF), jnp.float32),
        mesh=plsc.VectorSubcoreMesh(core_axis_name="c", subcore_axis_name="s"),
        scratch_types=[
            pltpu.VMEM_SHARED((ACC_ROWS, HALF), jnp.float32),
            pltpu.VMEM((CHUNKS_PER_TILE, CHUNK), jnp.int32),
            pltpu.VMEM((CHUNKS_PER_TILE, CHUNK), jnp.int32),
            pltpu.VMEM((CHUNK, HALF), jnp.float32),
            pltpu.SemaphoreType.DMA,
        ],
    )
    def k(feat_hbm, src_hbm, dst_hbm, h_hbm, acc, src_v, dst_v, rows_v, sem):
        c = lax.axis_index("c")
        s = lax.axis_index("s")
        row0 = s * ROWS_PER_TILE
        pltpu.sync_copy(
            feat_hbm.at[pl.ds(c * N_PAD + row0, ROWS_PER_TILE)],
            acc.at[pl.ds(row0, ROWS_PER_TILE)],
        )
        pltpu.sync_copy(src_hbm.at[c, s], src_v)
        pltpu.sync_copy(dst_hbm.at[s], dst_v)
        plsc.subcore_barrier()

        @pl.loop(0, CHUNKS_PER_TILE)
        def _(j):
            pltpu.async_copy(feat_hbm.at[src_v.at[j]], rows_v, sem).wait()
            pltpu.sync_copy(rows_v, acc.at[dst_v.at[j]], add=True)

        plsc.subcore_barrier()
        pltpu.sync_copy(
            acc.at[pl.ds(row0, ROWS_PER_TILE)],
            h_hbm.at[c, pl.ds(row0, ROWS_PER_TILE)],
        )

    return k(feat_cat, src2, dst_r)


ROW_BLK = 1000


def _mm_body(h0_ref, h1_ref, wl_ref, wr_ref, b_ref, o_ref):
    o_ref[...] = (
        jnp.dot(h0_ref[0], wl_ref[...], preferred_element_type=jnp.float32,
                precision=lax.Precision.HIGHEST)
        + jnp.dot(h1_ref[0], wr_ref[...], preferred_element_type=jnp.float32,
                  precision=lax.Precision.HIGHEST)
        + b_ref[...]
    )


def _tc_linear(h2, W, b):
    wl = W[:, :HALF].T
    wr = W[:, HALF:].T
    b2 = b.reshape(1, D_OUT)
    return pl.pallas_call(
        _mm_body,
        grid=(N_NODES // ROW_BLK,),
        in_specs=[
            pl.BlockSpec((1, ROW_BLK, HALF), lambda i: (0, i, 0)),
            pl.BlockSpec((1, ROW_BLK, HALF), lambda i: (1, i, 0)),
            pl.BlockSpec((HALF, D_OUT), lambda i: (0, 0)),
            pl.BlockSpec((HALF, D_OUT), lambda i: (0, 0)),
            pl.BlockSpec((1, D_OUT), lambda i: (0, 0)),
        ],
        out_specs=pl.BlockSpec((ROW_BLK, D_OUT), lambda i: (i, 0)),
        out_shape=jax.ShapeDtypeStruct((N_NODES, D_OUT), jnp.float32),
    )(h2, h2, wl, wr, b2)


@jax.jit
def kernel(feature, edge_index, W, b):
    src = edge_index[0].astype(jnp.int32)
    dst = edge_index[1].astype(jnp.int32)
    pad = E_PAD - N_EDGES
    src_p = jnp.concatenate(
        [src, jnp.arange(pad, dtype=jnp.int32) % N_NODES])
    dst_p = jnp.concatenate(
        [dst, N_NODES + (jnp.arange(pad, dtype=jnp.int32) % (N_PAD - N_NODES))])
    src_r = src_p.reshape(NS, CHUNKS_PER_TILE, CHUNK)
    src2 = jnp.stack([src_r, src_r + N_PAD])
    dst_r = dst_p.reshape(NS, CHUNKS_PER_TILE, CHUNK)
    feat_pad = jnp.concatenate(
        [feature, jnp.zeros((N_PAD - N_NODES, D_IN), jnp.float32)])
    feat_cat = feat_pad.reshape(N_PAD, NC, HALF).transpose(1, 0, 2).reshape(
        NC * N_PAD, HALF)

    h2 = _sc_scatter(feat_cat, src2, dst_r)
    return _tc_linear(h2, W, b)


# speedup vs baseline: 3.1209x; 1.0372x over previous
import functools

import jax
import jax.numpy as jnp
from jax import lax
from jax.experimental import pallas as pl
from jax.experimental.pallas import tpu as pltpu
from jax.experimental.pallas import tpu_sc as plsc

N_NODES = 10000
N_EDGES = 160000
D_IN = 256
D_OUT = 256

HALF = D_IN // 2
NC = 2
NS = 16
CHUNK = 128
CHUNKS_PER_TILE = 79
EDGES_PER_TILE = CHUNKS_PER_TILE * CHUNK
E_PAD = NS * EDGES_PER_TILE
ROWS_PER_TILE = 632
N_PAD = NS * ROWS_PER_TILE
ACC_ROWS = N_PAD


def _sc_scatter(feat_cat, src2, dst_r):
    @functools.partial(
        pl.kernel,
        out_type=jax.ShapeDtypeStruct((NC, N_PAD, HALF), jnp.float32),
        mesh=plsc.VectorSubcoreMesh(core_axis_name="c", subcore_axis_name="s"),
        scratch_types=[
            pltpu.VMEM_SHARED((ACC_ROWS, HALF), jnp.float32),
            pltpu.VMEM((CHUNKS_PER_TILE, CHUNK), jnp.int32),
            pltpu.VMEM((CHUNKS_PER_TILE, CHUNK), jnp.int32),
            pltpu.VMEM((CHUNK, HALF), jnp.float32),
            pltpu.SemaphoreType.DMA,
        ],
    )
    def k(feat_hbm, src_hbm, dst_hbm, h_hbm, acc, src_v, dst_v, rows_v, sem):
        c = lax.axis_index("c")
        s = lax.axis_index("s")
        row0 = s * ROWS_PER_TILE
        pltpu.sync_copy(
            feat_hbm.at[pl.ds(c * N_PAD + row0, ROWS_PER_TILE)],
            acc.at[pl.ds(row0, ROWS_PER_TILE)],
        )
        pltpu.sync_copy(src_hbm.at[c, s], src_v)
        pltpu.sync_copy(dst_hbm.at[s], dst_v)
        plsc.subcore_barrier()

        @pl.loop(0, CHUNKS_PER_TILE)
        def _(j):
            pltpu.async_copy(feat_hbm.at[src_v.at[j]], rows_v, sem).wait()
            pltpu.sync_copy(rows_v, acc.at[dst_v.at[j]], add=True)

        plsc.subcore_barrier()
        pltpu.sync_copy(
            acc.at[pl.ds(row0, ROWS_PER_TILE)],
            h_hbm.at[c, pl.ds(row0, ROWS_PER_TILE)],
        )

    return k(feat_cat, src2, dst_r)


ROW_BLK = 1000


def _mm_body(h0_ref, h1_ref, wl_ref, wr_ref, b_ref, o_ref):
    o_ref[...] = (
        jnp.dot(h0_ref[0], wl_ref[...], preferred_element_type=jnp.float32,
                precision=lax.Precision.HIGHEST)
        + jnp.dot(h1_ref[0], wr_ref[...], preferred_element_type=jnp.float32,
                  precision=lax.Precision.HIGHEST)
        + b_ref[...]
    )


def _tc_linear(h2, W, b):
    wl = W[:, :HALF].T
    wr = W[:, HALF:].T
    b2 = b.reshape(1, D_OUT)
    return pl.pallas_call(
        _mm_body,
        grid=(N_NODES // ROW_BLK,),
        in_specs=[
            pl.BlockSpec((1, ROW_BLK, HALF), lambda i: (0, i, 0)),
            pl.BlockSpec((1, ROW_BLK, HALF), lambda i: (1, i, 0)),
            pl.BlockSpec((HALF, D_OUT), lambda i: (0, 0)),
            pl.BlockSpec((HALF, D_OUT), lambda i: (0, 0)),
            pl.BlockSpec((1, D_OUT), lambda i: (0, 0)),
        ],
        out_specs=pl.BlockSpec((ROW_BLK, D_OUT), lambda i: (i, 0)),
        out_shape=jax.ShapeDtypeStruct((N_NODES, D_OUT), jnp.float32),
    )(h2, h2, wl, wr, b2)


@jax.jit
def kernel(feature, edge_index, W, b):
    src = edge_index[0].astype(jnp.int32)
    dst = edge_index[1].astype(jnp.int32)
    pad = E_PAD - N_EDGES
    src_p = jnp.concatenate(
        [src, jnp.arange(pad, dtype=jnp.int32) % N_NODES])
    dst_p = jnp.concatenate(
        [dst, N_NODES + (jnp.arange(pad, dtype=jnp.int32) % (N_PAD - N_NODES))])
    src_r = src_p.reshape(NS, CHUNKS_PER_TILE, CHUNK)
    src2 = jnp.stack([src_r, src_r + N_PAD])
    dst_r = dst_p.reshape(NS, CHUNKS_PER_TILE, CHUNK)
    feat_pad = jnp.concatenate(
        [feature, jnp.zeros((N_PAD - N_NODES, D_IN), jnp.float32)])
    feat_cat = feat_pad.reshape(N_PAD, NC, HALF).transpose(1, 0, 2).reshape(
        NC * N_PAD, HALF)

    h2 = _sc_scatter(feat_cat, src2, dst_r)
    return _tc_linear(h2, W, b)


# speedup vs baseline: 4.2336x; 1.3565x over previous
import functools

import jax
import jax.numpy as jnp
from jax import lax
from jax.experimental import pallas as pl
from jax.experimental.pallas import tpu as pltpu
from jax.experimental.pallas import tpu_sc as plsc

N_NODES = 10000
N_EDGES = 160000
D_IN = 256
D_OUT = 256

HALF = D_IN // 2
NC = 2
NS = 16
CHUNK = 128
CHUNKS_PER_TILE = 80
BLK = 20
NBLK = CHUNKS_PER_TILE // BLK
EDGES_PER_TILE = CHUNKS_PER_TILE * CHUNK
E_PAD = NS * EDGES_PER_TILE
ROWS_PER_TILE = 632
N_PAD = NS * ROWS_PER_TILE
ACC_ROWS = N_PAD


def _sc_scatter(feat_cat, src2, dst_r):
    @functools.partial(
        pl.kernel,
        out_type=jax.ShapeDtypeStruct((NC, N_PAD, HALF), jnp.float32),
        mesh=plsc.VectorSubcoreMesh(core_axis_name="c", subcore_axis_name="s"),
        scratch_types=[
            pltpu.VMEM_SHARED((ACC_ROWS, HALF), jnp.float32),
            pltpu.VMEM((BLK, CHUNK), jnp.int32),
            pltpu.VMEM((BLK, CHUNK), jnp.int32),
            pltpu.VMEM((CHUNKS_PER_TILE, CHUNK), jnp.int32),
            pltpu.VMEM((CHUNK, HALF), jnp.float32),
            pltpu.VMEM((CHUNK, HALF), jnp.float32),
            pltpu.SemaphoreType.DMA,
            pltpu.SemaphoreType.DMA,
            pltpu.SemaphoreType.DMA,
            pltpu.SemaphoreType.DMA,
        ],
    )
    def k(feat_hbm, src_hbm, dst_hbm, h_hbm, acc, src_0, src_1, dst_v,
          rows_a, rows_b, sem_a, sem_b, sem_i0, sem_i1):
        src_bufs = (src_0, src_1)
        idx_sems = (sem_i0, sem_i1)
        c = lax.axis_index("c")
        s = lax.axis_index("s")
        row0 = s * ROWS_PER_TILE
        pltpu.sync_copy(
            feat_hbm.at[pl.ds(c * N_PAD + row0, ROWS_PER_TILE)],
            acc.at[pl.ds(row0, ROWS_PER_TILE)],
        )
        pltpu.sync_copy(dst_hbm.at[s], dst_v)
        pltpu.sync_copy(src_hbm.at[c, s, 0], src_0)
        pltpu.async_copy(src_hbm.at[c, s, 1], src_1, sem_i1)
        plsc.subcore_barrier()

        pltpu.async_copy(feat_hbm.at[src_0.at[0]], rows_a, sem_a)

        for blk in range(NBLK):
            ib = src_bufs[blk % 2]
            base = blk * BLK

            @pl.loop(0, BLK, step=2)
            def _(jj):
                pltpu.async_copy(feat_hbm.at[ib.at[jj + 1]], rows_b, sem_b)
                pltpu.make_async_copy(
                    feat_hbm.at[ib.at[jj]], rows_a, sem_a).wait()
                pltpu.sync_copy(rows_a, acc.at[dst_v.at[base + jj]], add=True)

                @pl.when(jj + 2 < BLK)
                def _():
                    pltpu.async_copy(feat_hbm.at[ib.at[jj + 2]], rows_a, sem_a)

                pltpu.make_async_copy(
                    feat_hbm.at[ib.at[jj + 1]], rows_b, sem_b).wait()
                pltpu.sync_copy(
                    rows_b, acc.at[dst_v.at[base + jj + 1]], add=True)

            if blk + 2 < NBLK:
                pltpu.async_copy(
                    src_hbm.at[c, s, blk + 2], ib, idx_sems[blk % 2])
            if blk + 1 < NBLK:
                nb = src_bufs[(blk + 1) % 2]
                pltpu.make_async_copy(
                    src_hbm.at[c, s, blk + 1], nb, idx_sems[(blk + 1) % 2]
                ).wait()
                pltpu.async_copy(feat_hbm.at[nb.at[0]], rows_a, sem_a)

        plsc.subcore_barrier()
        pltpu.sync_copy(
            acc.at[pl.ds(row0, ROWS_PER_TILE)],
            h_hbm.at[c, pl.ds(row0, ROWS_PER_TILE)],
        )

    return k(feat_cat, src2, dst_r)


ROW_BLK = 1000


def _mm_body(h0_ref, h1_ref, wl_ref, wr_ref, b_ref, o_ref):
    o_ref[...] = (
        jnp.dot(h0_ref[0], wl_ref[...], preferred_element_type=jnp.float32,
                precision=lax.Precision.HIGHEST)
        + jnp.dot(h1_ref[0], wr_ref[...], preferred_element_type=jnp.float32,
                  precision=lax.Precision.HIGHEST)
        + b_ref[...]
    )


def _tc_linear(h2, W, b):
    wl = W[:, :HALF].T
    wr = W[:, HALF:].T
    b2 = b.reshape(1, D_OUT)
    return pl.pallas_call(
        _mm_body,
        grid=(N_NODES // ROW_BLK,),
        in_specs=[
            pl.BlockSpec((1, ROW_BLK, HALF), lambda i: (0, i, 0)),
            pl.BlockSpec((1, ROW_BLK, HALF), lambda i: (1, i, 0)),
            pl.BlockSpec((HALF, D_OUT), lambda i: (0, 0)),
            pl.BlockSpec((HALF, D_OUT), lambda i: (0, 0)),
            pl.BlockSpec((1, D_OUT), lambda i: (0, 0)),
        ],
        out_specs=pl.BlockSpec((ROW_BLK, D_OUT), lambda i: (i, 0)),
        out_shape=jax.ShapeDtypeStruct((N_NODES, D_OUT), jnp.float32),
    )(h2, h2, wl, wr, b2)


@jax.jit
def kernel(feature, edge_index, W, b):
    src = edge_index[0].astype(jnp.int32)
    dst = edge_index[1].astype(jnp.int32)
    pad = E_PAD - N_EDGES
    src_p = jnp.concatenate(
        [src, jnp.arange(pad, dtype=jnp.int32) % N_NODES])
    dst_p = jnp.concatenate(
        [dst, N_NODES + (jnp.arange(pad, dtype=jnp.int32) % (N_PAD - N_NODES))])
    src_r = src_p.reshape(NS, NBLK, BLK, CHUNK)
    src2 = jnp.stack([src_r, src_r + N_PAD])
    dst_r = dst_p.reshape(NS, CHUNKS_PER_TILE, CHUNK)
    feat_pad = jnp.concatenate(
        [feature, jnp.zeros((N_PAD - N_NODES, D_IN), jnp.float32)])
    feat_cat = feat_pad.reshape(N_PAD, NC, HALF).transpose(1, 0, 2).reshape(
        NC * N_PAD, HALF)

    h2 = _sc_scatter(feat_cat, src2, dst_r)
    return _tc_linear(h2, W, b)
